# initial kernel scaffold (unmeasured)
import jax
import jax.numpy as jnp
from jax import lax
from jax.experimental import pallas as pl
from jax.experimental.pallas import tpu as pltpu

S = 2048
S_HALF = 1024
K = 4096
N = 8192
BN = 1024
NJ = N // BN


def kernel(O, Wo):
    A = O.reshape(S, K)

    def body(a_ref, w_ref, out_ref, send_buf, recv_buf, send_sems, recv_sems):
        j = pl.program_id(0)
        my_x = lax.axis_index("x")
        my_y = lax.axis_index("y")
        my_z = lax.axis_index("z")
        keep_start = my_x * S_HALF
        send_start = (1 - my_x) * S_HALF
        slot = j % 2

        send_buf[slot] = jnp.dot(
            a_ref[pl.ds(send_start, S_HALF), :],
            w_ref[...],
            preferred_element_type=jnp.float32,
        )

        rdma = pltpu.make_async_remote_copy(
            src_ref=send_buf.at[slot],
            dst_ref=recv_buf.at[slot],
            send_sem=send_sems.at[slot],
            recv_sem=recv_sems.at[slot],
            device_id=(1 - my_x, my_y, my_z),
            device_id_type=pl.DeviceIdType.MESH,
        )
        rdma.start()

        keep = jnp.dot(
            a_ref[pl.ds(keep_start, S_HALF), :],
            w_ref[...],
            preferred_element_type=jnp.float32,
        )

        rdma.wait()
        out_ref[...] = keep + recv_buf[slot]

    out = pl.pallas_call(
        body,
        grid=(NJ,),
        in_specs=[
            pl.BlockSpec((S, K), lambda j: (0, 0), memory_space=pltpu.VMEM),
            pl.BlockSpec((K, BN), lambda j: (0, j), memory_space=pltpu.VMEM),
        ],
        out_specs=pl.BlockSpec(
            (S_HALF, BN), lambda j: (0, j), memory_space=pltpu.VMEM
        ),
        out_shape=jax.ShapeDtypeStruct((S_HALF, N), jnp.float32),
        scratch_shapes=[
            pltpu.VMEM((2, S_HALF, BN), jnp.float32),
            pltpu.VMEM((2, S_HALF, BN), jnp.float32),
            pltpu.SemaphoreType.DMA((2,)),
            pltpu.SemaphoreType.DMA((2,)),
        ],
        compiler_params=pltpu.CompilerParams(collective_id=0),
    )(A, Wo)
    return out.reshape(1, S_HALF, N)


# baseline (device time: 697013 ns/iter reference)
import jax
import jax.numpy as jnp
from jax import lax
from jax.experimental import pallas as pl
from jax.experimental.pallas import tpu as pltpu

S = 2048
S_HALF = 1024
K = 4096
N = 8192
BN = 512
BK = 1024
NJ = N // BN
NK = K // BK


def kernel(O, Wo):
    A = O.reshape(S, K)

    def body(a_ref, w_ref, out_ref, acc, send_buf, recv_buf, send_sems, recv_sems):
        j = pl.program_id(0)
        k = pl.program_id(1)
        my_x = lax.axis_index("x")
        my_y = lax.axis_index("y")
        my_z = lax.axis_index("z")
        keep_start = my_x * S_HALF
        send_start = (1 - my_x) * S_HALF
        slot = j % 2

        prod = jnp.dot(a_ref[...], w_ref[...], preferred_element_type=jnp.float32)

        @pl.when(k == 0)
        def _():
            acc[...] = prod

        @pl.when(k != 0)
        def _():
            acc[...] += prod

        @pl.when(k == NK - 1)
        def _():
            send_buf[slot] = acc[pl.ds(send_start, S_HALF), :]
            rdma = pltpu.make_async_remote_copy(
                src_ref=send_buf.at[slot],
                dst_ref=recv_buf.at[slot],
                send_sem=send_sems.at[slot],
                recv_sem=recv_sems.at[slot],
                device_id=(1 - my_x, my_y, my_z),
                device_id_type=pl.DeviceIdType.MESH,
            )
            rdma.start()
            rdma.wait()
            out_ref[...] = acc[pl.ds(keep_start, S_HALF), :] + recv_buf[slot]

    out = pl.pallas_call(
        body,
        grid=(NJ, NK),
        in_specs=[
            pl.BlockSpec((S, BK), lambda j, k: (0, k), memory_space=pltpu.VMEM),
            pl.BlockSpec((BK, BN), lambda j, k: (k, j), memory_space=pltpu.VMEM),
        ],
        out_specs=pl.BlockSpec(
            (S_HALF, BN), lambda j, k: (0, j), memory_space=pltpu.VMEM
        ),
        out_shape=jax.ShapeDtypeStruct((S_HALF, N), jnp.float32),
        scratch_shapes=[
            pltpu.VMEM((S, BN), jnp.float32),
            pltpu.VMEM((2, S_HALF, BN), jnp.float32),
            pltpu.VMEM((2, S_HALF, BN), jnp.float32),
            pltpu.SemaphoreType.DMA((2,)),
            pltpu.SemaphoreType.DMA((2,)),
        ],
        compiler_params=pltpu.CompilerParams(
            vmem_limit_bytes=60 * 1024 * 1024,
        ),
    )(A, Wo)
    return out.reshape(1, S_HALF, N)


# device time: 471726 ns/iter; 1.4776x vs baseline; 1.4776x over previous
import jax
import jax.numpy as jnp
from jax import lax
from jax.experimental import pallas as pl
from jax.experimental.pallas import tpu as pltpu

S = 2048
S_HALF = 1024
K = 4096
N = 8192
BN = 512
BK = 1024
NJ = N // BN
NK = K // BK


def kernel(O, Wo):
    A = O.reshape(S, K)

    def body(a_ref, w_ref, out_ref, acc, keep_buf, send_buf, recv_buf,
             send_sems, recv_sems):
        j = pl.program_id(0)
        k = pl.program_id(1)
        my_x = lax.axis_index("x")
        my_y = lax.axis_index("y")
        my_z = lax.axis_index("z")
        keep_start = my_x * S_HALF
        send_start = (1 - my_x) * S_HALF
        partner = (1 - my_x, my_y, my_z)
        s = j % 2
        p = (j + 1) % 2

        def exchange(slot):
            return pltpu.make_async_remote_copy(
                src_ref=send_buf.at[slot],
                dst_ref=recv_buf.at[slot],
                send_sem=send_sems.at[slot],
                recv_sem=recv_sems.at[slot],
                device_id=partner,
                device_id_type=pl.DeviceIdType.MESH,
            )

        @pl.when(j < NJ)
        def _():
            prod = jnp.dot(
                a_ref[...], w_ref[...], preferred_element_type=jnp.float32
            )

            @pl.when(k == 0)
            def _():
                acc[...] = prod

            @pl.when(k != 0)
            def _():
                acc[...] += prod

        @pl.when(k == NK - 1)
        def _():
            @pl.when(j > 0)
            def _():
                exchange(p).wait_recv()
                out_ref[...] = keep_buf[p] + recv_buf[p]

            @pl.when(j < NJ)
            def _():
                @pl.when(j >= 2)
                def _():
                    exchange(s).wait_send()

                send_buf[s] = acc[pl.ds(send_start, S_HALF), :]
                keep_buf[s] = acc[pl.ds(keep_start, S_HALF), :]
                exchange(s).start()

            @pl.when(j == NJ)
            def _():
                exchange(0).wait_send()
                exchange(1).wait_send()

    out = pl.pallas_call(
        body,
        grid=(NJ + 1, NK),
        in_specs=[
            pl.BlockSpec((S, BK), lambda j, k: (0, k), memory_space=pltpu.VMEM),
            pl.BlockSpec(
                (BK, BN),
                lambda j, k: (k, jnp.minimum(j, NJ - 1)),
                memory_space=pltpu.VMEM,
            ),
        ],
        out_specs=pl.BlockSpec(
            (S_HALF, BN),
            lambda j, k: (0, jnp.maximum(j - 1, 0)),
            memory_space=pltpu.VMEM,
        ),
        out_shape=jax.ShapeDtypeStruct((S_HALF, N), jnp.float32),
        scratch_shapes=[
            pltpu.VMEM((S, BN), jnp.float32),
            pltpu.VMEM((2, S_HALF, BN), jnp.float32),
            pltpu.VMEM((2, S_HALF, BN), jnp.float32),
            pltpu.VMEM((2, S_HALF, BN), jnp.float32),
            pltpu.SemaphoreType.DMA((2,)),
            pltpu.SemaphoreType.DMA((2,)),
        ],
        compiler_params=pltpu.CompilerParams(
            vmem_limit_bytes=60 * 1024 * 1024,
        ),
    )(A, Wo)
    return out.reshape(1, S_HALF, N)


# device time: 455522 ns/iter; 1.5301x vs baseline; 1.0356x over previous
import jax
import jax.numpy as jnp
from jax import lax
from jax.experimental import pallas as pl
from jax.experimental.pallas import tpu as pltpu

S = 2048
S_HALF = 1024
K = 4096
N = 8192
BN = 512
BK = 1024
NJ = N // BN
NK = K // BK


def kernel(O, Wo):
    A = O.reshape(S, K)

    def body(a_ref, w_ref, out_ref, keep_buf, send_buf, recv_buf,
             send_sems, recv_sems, credit_sems):
        j = pl.program_id(0)
        k = pl.program_id(1)
        my_x = lax.axis_index("x")
        my_y = lax.axis_index("y")
        my_z = lax.axis_index("z")
        keep_start = my_x * S_HALF
        send_start = (1 - my_x) * S_HALF
        partner = (1 - my_x, my_y, my_z)
        s = j % 2
        p = (j + 1) % 2

        def exchange(slot):
            return pltpu.make_async_remote_copy(
                src_ref=send_buf.at[slot],
                dst_ref=recv_buf.at[slot],
                send_sem=send_sems.at[slot],
                recv_sem=recv_sems.at[slot],
                device_id=partner,
                device_id_type=pl.DeviceIdType.MESH,
            )

        @pl.when(j < NJ)
        def _():
            @pl.when((k == 0) & (j >= 2))
            def _():
                exchange(s).wait_send()
                pl.semaphore_wait(credit_sems.at[s], 1)

            prod_send = jnp.dot(
                a_ref[pl.ds(send_start, S_HALF), :],
                w_ref[...],
                preferred_element_type=jnp.float32,
            )
            prod_keep = jnp.dot(
                a_ref[pl.ds(keep_start, S_HALF), :],
                w_ref[...],
                preferred_element_type=jnp.float32,
            )

            @pl.when(k == 0)
            def _():
                send_buf[s] = prod_send
                keep_buf[s] = prod_keep

            @pl.when(k != 0)
            def _():
                send_buf[s] += prod_send
                keep_buf[s] += prod_keep

            @pl.when(k == NK - 1)
            def _():
                exchange(s).start()

        @pl.when(k == NK - 1)
        def _():
            @pl.when(j > 0)
            def _():
                exchange(p).wait_recv()
                out_ref[...] = keep_buf[p] + recv_buf[p]

                @pl.when(j < NJ - 1)
                def _():
                    pl.semaphore_signal(
                        credit_sems.at[p],
                        inc=1,
                        device_id=partner,
                        device_id_type=pl.DeviceIdType.MESH,
                    )

            @pl.when(j == NJ)
            def _():
                exchange(0).wait_send()
                exchange(1).wait_send()

    out = pl.pallas_call(
        body,
        grid=(NJ + 1, NK),
        in_specs=[
            pl.BlockSpec((S, BK), lambda j, k: (0, k), memory_space=pltpu.VMEM),
            pl.BlockSpec(
                (BK, BN),
                lambda j, k: (k, jnp.minimum(j, NJ - 1)),
                memory_space=pltpu.VMEM,
            ),
        ],
        out_specs=pl.BlockSpec(
            (S_HALF, BN),
            lambda j, k: (0, jnp.maximum(j - 1, 0)),
            memory_space=pltpu.VMEM,
        ),
        out_shape=jax.ShapeDtypeStruct((S_HALF, N), jnp.float32),
        scratch_shapes=[
            pltpu.VMEM((2, S_HALF, BN), jnp.float32),
            pltpu.VMEM((2, S_HALF, BN), jnp.float32),
            pltpu.VMEM((2, S_HALF, BN), jnp.float32),
            pltpu.SemaphoreType.DMA((2,)),
            pltpu.SemaphoreType.DMA((2,)),
            pltpu.SemaphoreType.REGULAR((2,)),
        ],
        compiler_params=pltpu.CompilerParams(
            vmem_limit_bytes=60 * 1024 * 1024,
        ),
    )(A, Wo)
    return out.reshape(1, S_HALF, N)


# device time: 425089 ns/iter; 1.6397x vs baseline; 1.0716x over previous
import jax
import jax.numpy as jnp
from jax import lax
from jax.experimental import pallas as pl
from jax.experimental.pallas import tpu as pltpu

S = 2048
S_HALF = 1024
H = 32
D = 128
K = H * D
N = 8192
BN = 512
BH = 8
BK = BH * D
NJ = N // BN
NK = H // BH


def kernel(O, Wo):
    def body(o_ref, w_ref, out_ref, keep_buf, send_buf, recv_buf,
             send_sems, recv_sems, credit_sems):
        j = pl.program_id(0)
        k = pl.program_id(1)
        my_x = lax.axis_index("x")
        my_y = lax.axis_index("y")
        my_z = lax.axis_index("z")
        keep_start = my_x * S_HALF
        send_start = (1 - my_x) * S_HALF
        partner = (1 - my_x, my_y, my_z)
        s = j % 2
        p = (j + 1) % 2

        def exchange(slot):
            return pltpu.make_async_remote_copy(
                src_ref=send_buf.at[slot],
                dst_ref=recv_buf.at[slot],
                send_sem=send_sems.at[slot],
                recv_sem=recv_sems.at[slot],
                device_id=partner,
                device_id_type=pl.DeviceIdType.MESH,
            )

        def half_partial(row_start):
            acc = None
            for h in range(BH):
                a = o_ref[0, pl.ds(row_start, S_HALF), h, :]
                w = w_ref[pl.ds(h * D, D), :]
                d = jnp.dot(a, w, preferred_element_type=jnp.float32)
                acc = d if acc is None else acc + d
            return acc

        @pl.when(j < NJ)
        def _():
            @pl.when((k == 0) & (j >= 2))
            def _():
                exchange(s).wait_send()
                pl.semaphore_wait(credit_sems.at[s], 1)

            prod_send = half_partial(send_start)
            prod_keep = half_partial(keep_start)

            @pl.when(k == 0)
            def _():
                send_buf[s] = prod_send
                keep_buf[s] = prod_keep

            @pl.when(k != 0)
            def _():
                send_buf[s] += prod_send
                keep_buf[s] += prod_keep

            @pl.when(k == NK - 1)
            def _():
                exchange(s).start()

        @pl.when(k == NK - 1)
        def _():
            @pl.when(j > 0)
            def _():
                exchange(p).wait_recv()
                out_ref[0, :, :] = keep_buf[p] + recv_buf[p]

                @pl.when(j < NJ - 1)
                def _():
                    pl.semaphore_signal(
                        credit_sems.at[p],
                        inc=1,
                        device_id=partner,
                        device_id_type=pl.DeviceIdType.MESH,
                    )

            @pl.when(j == NJ)
            def _():
                exchange(0).wait_send()
                exchange(1).wait_send()

    out = pl.pallas_call(
        body,
        grid=(NJ + 1, NK),
        in_specs=[
            pl.BlockSpec(
                (1, S, BH, D),
                lambda j, k: (0, 0, k, 0),
                memory_space=pltpu.VMEM,
            ),
            pl.BlockSpec(
                (BK, BN),
                lambda j, k: (k, jnp.minimum(j, NJ - 1)),
                memory_space=pltpu.VMEM,
            ),
        ],
        out_specs=pl.BlockSpec(
            (1, S_HALF, BN),
            lambda j, k: (0, 0, jnp.maximum(j - 1, 0)),
            memory_space=pltpu.VMEM,
        ),
        out_shape=jax.ShapeDtypeStruct((1, S_HALF, N), jnp.float32),
        scratch_shapes=[
            pltpu.VMEM((2, S_HALF, BN), jnp.float32),
            pltpu.VMEM((2, S_HALF, BN), jnp.float32),
            pltpu.VMEM((2, S_HALF, BN), jnp.float32),
            pltpu.SemaphoreType.DMA((2,)),
            pltpu.SemaphoreType.DMA((2,)),
            pltpu.SemaphoreType.REGULAR((2,)),
        ],
        compiler_params=pltpu.CompilerParams(
            vmem_limit_bytes=60 * 1024 * 1024,
        ),
    )(O, Wo)
    return out


# device time: 269821 ns/iter; 2.5832x vs baseline; 1.5754x over previous
import jax
import jax.numpy as jnp
from jax import lax
from jax.experimental import pallas as pl
from jax.experimental.pallas import tpu as pltpu

S = 2048
S_HALF = 1024
H = 32
D = 128
K = H * D
N = 8192
N_HALF = N // 2
BN = 512
BH = 8
BK = BH * D
NJ = N_HALF // BN
NK = H // BH


def kernel(O, Wo):
    q0 = jnp.full((1,), lax.axis_index("y"), dtype=jnp.int32)

    def body(q_ref, o_ref, w_ref, out_ref, keep, send_x, recv_x, red, recv_y,
             x_send_sems, x_recv_sems, x_credits,
             y_send_sems, y_recv_sems, y_credits,
             outred_sems, outy_sems):
        b = pl.program_id(0)
        k = pl.program_id(1)
        my_x = lax.axis_index("x")
        my_y = lax.axis_index("y")
        my_z = lax.axis_index("z")
        keep_start = my_x * S_HALF
        send_start = (1 - my_x) * S_HALF
        x_partner = (1 - my_x, my_y, my_z)
        y_partner = (my_x, 1 - my_y, my_z)
        my_col0 = my_y * N_HALF
        other_col0 = (1 - my_y) * N_HALF
        s = b % 2
        p = (b + 1) % 2

        def x_exchange(slot):
            return pltpu.make_async_remote_copy(
                src_ref=send_x.at[slot],
                dst_ref=recv_x.at[slot],
                send_sem=x_send_sems.at[slot],
                recv_sem=x_recv_sems.at[slot],
                device_id=x_partner,
                device_id_type=pl.DeviceIdType.MESH,
            )

        def y_exchange(slot):
            return pltpu.make_async_remote_copy(
                src_ref=red.at[slot],
                dst_ref=recv_y.at[slot],
                send_sem=y_send_sems.at[slot],
                recv_sem=y_recv_sems.at[slot],
                device_id=y_partner,
                device_id_type=pl.DeviceIdType.MESH,
            )

        def half_partial(row_start):
            acc = None
            for h in range(BH):
                a = o_ref[0, pl.ds(row_start, S_HALF), h, :]
                w = w_ref[pl.ds(h * D, D), :]
                d = jnp.dot(a, w, preferred_element_type=jnp.float32)
                acc = d if acc is None else acc + d
            return acc

        @pl.when(b < NJ)
        def _():
            @pl.when((k == 0) & (b >= 2))
            def _():
                x_exchange(s).wait_send()
                pl.semaphore_wait(x_credits.at[s], 1)

            prod_send = half_partial(send_start)
            prod_keep = half_partial(keep_start)

            @pl.when(k == 0)
            def _():
                send_x[s] = prod_send
                keep[s] = prod_keep

            @pl.when(k != 0)
            def _():
                send_x[s] += prod_send
                keep[s] += prod_keep

            @pl.when(k == NK - 1)
            def _():
                x_exchange(s).start()

        @pl.when(k == NK - 1)
        def _():
            @pl.when((b >= 1) & (b <= NJ))
            def _():
                c = b - 1
                x_exchange(p).wait_recv()

                @pl.when(b >= 3)
                def _():
                    pltpu.make_async_copy(
                        red.at[p], red.at[p], outred_sems.at[p]
                    ).wait()

                @pl.when(c >= 2)
                def _():
                    y_exchange(p).wait_send()
                    pl.semaphore_wait(y_credits.at[p], 1)

                red[p] = keep[p] + recv_x[p]

                @pl.when(b <= NJ - 2)
                def _():
                    pl.semaphore_signal(
                        x_credits.at[p],
                        inc=1,
                        device_id=x_partner,
                        device_id_type=pl.DeviceIdType.MESH,
                    )

                y_exchange(p).start()
                pltpu.make_async_copy(
                    red.at[p],
                    out_ref.at[0, :, pl.ds(my_col0 + c * BN, BN)],
                    outred_sems.at[p],
                ).start()

            @pl.when(b >= 2)
            def _():
                c2 = b - 2
                y_exchange(s).wait_recv()
                cp = pltpu.make_async_copy(
                    recv_y.at[s],
                    out_ref.at[0, :, pl.ds(other_col0 + c2 * BN, BN)],
                    outy_sems.at[s],
                )
                cp.start()
                cp.wait()

                @pl.when(b <= NJ + 1 - 2)
                def _():
                    pl.semaphore_signal(
                        y_credits.at[s],
                        inc=1,
                        device_id=y_partner,
                        device_id_type=pl.DeviceIdType.MESH,
                    )

            @pl.when(b == NJ + 1)
            def _():
                x_exchange(0).wait_send()
                x_exchange(1).wait_send()
                y_exchange(0).wait_send()
                y_exchange(1).wait_send()
                pltpu.make_async_copy(
                    red.at[0], red.at[0], outred_sems.at[0]
                ).wait()
                pltpu.make_async_copy(
                    red.at[1], red.at[1], outred_sems.at[1]
                ).wait()

    grid_spec = pltpu.PrefetchScalarGridSpec(
        num_scalar_prefetch=1,
        grid=(NJ + 2, NK),
        in_specs=[
            pl.BlockSpec(
                (1, S, BH, D),
                lambda b, k, q_ref: (0, 0, k, 0),
            ),
            pl.BlockSpec(
                (BK, BN),
                lambda b, k, q_ref: (
                    k,
                    q_ref[0] * NJ + jnp.minimum(b, NJ - 1),
                ),
            ),
        ],
        out_specs=pl.BlockSpec(memory_space=pl.ANY),
        scratch_shapes=[
            pltpu.VMEM((2, S_HALF, BN), jnp.float32),
            pltpu.VMEM((2, S_HALF, BN), jnp.float32),
            pltpu.VMEM((2, S_HALF, BN), jnp.float32),
            pltpu.VMEM((2, S_HALF, BN), jnp.float32),
            pltpu.VMEM((2, S_HALF, BN), jnp.float32),
            pltpu.SemaphoreType.DMA((2,)),
            pltpu.SemaphoreType.DMA((2,)),
            pltpu.SemaphoreType.REGULAR((2,)),
            pltpu.SemaphoreType.DMA((2,)),
            pltpu.SemaphoreType.DMA((2,)),
            pltpu.SemaphoreType.REGULAR((2,)),
            pltpu.SemaphoreType.DMA((2,)),
            pltpu.SemaphoreType.DMA((2,)),
        ],
    )

    out = pl.pallas_call(
        body,
        grid_spec=grid_spec,
        out_shape=jax.ShapeDtypeStruct((1, S_HALF, N), jnp.float32),
        compiler_params=pltpu.CompilerParams(
            vmem_limit_bytes=60 * 1024 * 1024,
        ),
    )(q0, O, Wo)
    return out


# device time: 262641 ns/iter; 2.6539x vs baseline; 1.0273x over previous
import jax
import jax.numpy as jnp
from jax import lax
from jax.experimental import pallas as pl
from jax.experimental.pallas import tpu as pltpu

S = 2048
S_HALF = 1024
H = 32
D = 128
K = H * D
N = 8192
NQ = N // 4
BN = 512
BH = 8
BK = BH * D
NJ = NQ // BN
NK = H // BH


def kernel(O, Wo):
    r0 = jnp.full(
        (1,), 2 * lax.axis_index("y") + lax.axis_index("z"), dtype=jnp.int32
    )

    def body(r_ref, o_ref, w_ref, out_ref, keep, send_x, recv_x,
             x_send_sems, x_recv_sems, y_send_sems, y_recv_sems,
             z1_send_sems, z1_recv_sems, z2_send_sems, z2_recv_sems,
             outred_sems):
        b = pl.program_id(0)
        k = pl.program_id(1)
        my_x = lax.axis_index("x")
        my_y = lax.axis_index("y")
        my_z = lax.axis_index("z")
        keep_start = my_x * S_HALF
        send_start = (1 - my_x) * S_HALF
        x_partner = (1 - my_x, my_y, my_z)
        y_partner = (my_x, 1 - my_y, my_z)
        z_partner = (my_x, my_y, 1 - my_z)
        r = 2 * my_y + my_z
        ry = 2 * (1 - my_y) + my_z
        rz = 2 * my_y + (1 - my_z)
        rd = 2 * (1 - my_y) + (1 - my_z)

        def out_slice(quarter, i):
            return out_ref.at[0, :, pl.ds(quarter * NQ + i * BN, BN)]

        def x_rdma(i):
            return pltpu.make_async_remote_copy(
                src_ref=send_x.at[i],
                dst_ref=recv_x.at[i],
                send_sem=x_send_sems.at[i],
                recv_sem=x_recv_sems.at[i],
                device_id=x_partner,
                device_id_type=pl.DeviceIdType.MESH,
            )

        def y_rdma(i):
            return pltpu.make_async_remote_copy(
                src_ref=keep.at[i],
                dst_ref=out_slice(r, i),
                send_sem=y_send_sems.at[i],
                recv_sem=y_recv_sems.at[i],
                device_id=y_partner,
                device_id_type=pl.DeviceIdType.MESH,
            )

        def y_wait(i):
            return pltpu.make_async_remote_copy(
                src_ref=keep.at[i],
                dst_ref=out_slice(ry, i),
                send_sem=y_send_sems.at[i],
                recv_sem=y_recv_sems.at[i],
                device_id=y_partner,
                device_id_type=pl.DeviceIdType.MESH,
            )

        def z1_rdma(i):
            return pltpu.make_async_remote_copy(
                src_ref=keep.at[i],
                dst_ref=out_slice(r, i),
                send_sem=z1_send_sems.at[i],
                recv_sem=z1_recv_sems.at[i],
                device_id=z_partner,
                device_id_type=pl.DeviceIdType.MESH,
            )

        def z1_wait(i):
            return pltpu.make_async_remote_copy(
                src_ref=keep.at[i],
                dst_ref=out_slice(rz, i),
                send_sem=z1_send_sems.at[i],
                recv_sem=z1_recv_sems.at[i],
                device_id=z_partner,
                device_id_type=pl.DeviceIdType.MESH,
            )

        def z2_rdma(i):
            return pltpu.make_async_remote_copy(
                src_ref=out_slice(ry, i),
                dst_ref=out_slice(ry, i),
                send_sem=z2_send_sems.at[i],
                recv_sem=z2_recv_sems.at[i],
                device_id=z_partner,
                device_id_type=pl.DeviceIdType.MESH,
            )

        def z2_wait(i):
            return pltpu.make_async_remote_copy(
                src_ref=out_slice(rd, i),
                dst_ref=out_slice(rd, i),
                send_sem=z2_send_sems.at[i],
                recv_sem=z2_recv_sems.at[i],
                device_id=z_partner,
                device_id_type=pl.DeviceIdType.MESH,
            )

        def half_partial(row_start):
            acc = None
            for h in range(BH):
                a = o_ref[0, pl.ds(row_start, S_HALF), h, :]
                w = w_ref[pl.ds(h * D, D), :]
                d = jnp.dot(a, w, preferred_element_type=jnp.float32)
                acc = d if acc is None else acc + d
            return acc

        @pl.when(b < NJ)
        def _():
            prod_send = half_partial(send_start)
            prod_keep = half_partial(keep_start)

            @pl.when(k == 0)
            def _():
                send_x[b] = prod_send
                keep[b] = prod_keep

            @pl.when(k != 0)
            def _():
                send_x[b] += prod_send
                keep[b] += prod_keep

            @pl.when(k == NK - 1)
            def _():
                x_rdma(b).start()

        @pl.when(k == NK - 1)
        def _():
            @pl.when((b >= 1) & (b <= NJ))
            def _():
                c = b - 1
                x_rdma(c).wait_recv()
                keep[c] += recv_x[c]
                y_rdma(c).start()
                z1_rdma(c).start()
                pltpu.make_async_copy(
                    keep.at[c], out_slice(r, c), outred_sems.at[c]
                ).start()

            @pl.when((b >= 2) & (b <= NJ + 1))
            def _():
                c2 = b - 2
                y_wait(c2).wait_recv()
                z2_rdma(c2).start()
                z1_wait(c2).wait_recv()

            @pl.when((b >= 3) & (b <= NJ + 2))
            def _():
                z2_wait(b - 3).wait_recv()

            @pl.when(b == NJ + 2)
            def _():
                for i in range(NJ):
                    x_rdma(i).wait_send()
                    y_rdma(i).wait_send()
                    z1_rdma(i).wait_send()
                    z2_rdma(i).wait_send()
                    pltpu.make_async_copy(
                        keep.at[i], out_slice(r, i), outred_sems.at[i]
                    ).wait()

    grid_spec = pltpu.PrefetchScalarGridSpec(
        num_scalar_prefetch=1,
        grid=(NJ + 3, NK),
        in_specs=[
            pl.BlockSpec(
                (1, S, BH, D),
                lambda b, k, r_ref: (0, 0, k, 0),
            ),
            pl.BlockSpec(
                (BK, BN),
                lambda b, k, r_ref: (
                    k,
                    r_ref[0] * NJ + jnp.minimum(b, NJ - 1),
                ),
            ),
        ],
        out_specs=pl.BlockSpec(memory_space=pl.ANY),
        scratch_shapes=[
            pltpu.VMEM((NJ, S_HALF, BN), jnp.float32),
            pltpu.VMEM((NJ, S_HALF, BN), jnp.float32),
            pltpu.VMEM((NJ, S_HALF, BN), jnp.float32),
            pltpu.SemaphoreType.DMA((NJ,)),
            pltpu.SemaphoreType.DMA((NJ,)),
            pltpu.SemaphoreType.DMA((NJ,)),
            pltpu.SemaphoreType.DMA((NJ,)),
            pltpu.SemaphoreType.DMA((NJ,)),
            pltpu.SemaphoreType.DMA((NJ,)),
            pltpu.SemaphoreType.DMA((NJ,)),
            pltpu.SemaphoreType.DMA((NJ,)),
            pltpu.SemaphoreType.DMA((NJ,)),
        ],
    )

    out = pl.pallas_call(
        body,
        grid_spec=grid_spec,
        out_shape=jax.ShapeDtypeStruct((1, S_HALF, N), jnp.float32),
        compiler_params=pltpu.CompilerParams(
            vmem_limit_bytes=60 * 1024 * 1024,
        ),
    )(r0, O, Wo)
    return out


# device time: 227422 ns/iter; 3.0648x vs baseline; 1.1549x over previous
import jax
import jax.numpy as jnp
from jax import lax
from jax.experimental import pallas as pl
from jax.experimental.pallas import tpu as pltpu

S = 2048
S_HALF = 1024
H = 32
D = 128
K = H * D
N = 8192
NQ = N // 4
BN = 512
BH = 8
BK = BH * D
NJ = NQ // BN
NK = H // BH


def kernel(O, Wo):
    r0 = jnp.full(
        (1,), 2 * lax.axis_index("y") + lax.axis_index("z"), dtype=jnp.int32
    )

    def body(r_ref, o_ref, w_ref, out_ref, keep, send_x, recv_x,
             x_send_sems, x_recv_sems, y_send_sems, y_recv_sems,
             z1_send_sems, z1_recv_sems, z2_send_sems, z2_recv_sems,
             y2_send_sems, y2_recv_sems, outred_sems):
        b = pl.program_id(0)
        k = pl.program_id(1)
        my_x = lax.axis_index("x")
        my_y = lax.axis_index("y")
        my_z = lax.axis_index("z")
        keep_start = my_x * S_HALF
        send_start = (1 - my_x) * S_HALF
        x_partner = (1 - my_x, my_y, my_z)
        y_partner = (my_x, 1 - my_y, my_z)
        z_partner = (my_x, my_y, 1 - my_z)
        r = 2 * my_y + my_z
        ry = 2 * (1 - my_y) + my_z
        rz = 2 * my_y + (1 - my_z)
        rd = 2 * (1 - my_y) + (1 - my_z)

        def out_slice(quarter, i):
            return out_ref.at[0, :, pl.ds(quarter * NQ + i * BN, BN)]

        def x_rdma(i):
            return pltpu.make_async_remote_copy(
                src_ref=send_x.at[i],
                dst_ref=recv_x.at[i],
                send_sem=x_send_sems.at[i],
                recv_sem=x_recv_sems.at[i],
                device_id=x_partner,
                device_id_type=pl.DeviceIdType.MESH,
            )

        def y_rdma(i):
            return pltpu.make_async_remote_copy(
                src_ref=keep.at[i],
                dst_ref=out_slice(r, i),
                send_sem=y_send_sems.at[i],
                recv_sem=y_recv_sems.at[i],
                device_id=y_partner,
                device_id_type=pl.DeviceIdType.MESH,
            )

        def y_wait(i):
            return pltpu.make_async_remote_copy(
                src_ref=keep.at[i],
                dst_ref=out_slice(ry, i),
                send_sem=y_send_sems.at[i],
                recv_sem=y_recv_sems.at[i],
                device_id=y_partner,
                device_id_type=pl.DeviceIdType.MESH,
            )

        def z1_rdma(i):
            return pltpu.make_async_remote_copy(
                src_ref=keep.at[i],
                dst_ref=out_slice(r, i),
                send_sem=z1_send_sems.at[i],
                recv_sem=z1_recv_sems.at[i],
                device_id=z_partner,
                device_id_type=pl.DeviceIdType.MESH,
            )

        def z1_wait(i):
            return pltpu.make_async_remote_copy(
                src_ref=keep.at[i],
                dst_ref=out_slice(rz, i),
                send_sem=z1_send_sems.at[i],
                recv_sem=z1_recv_sems.at[i],
                device_id=z_partner,
                device_id_type=pl.DeviceIdType.MESH,
            )

        def z2_rdma(i):
            return pltpu.make_async_remote_copy(
                src_ref=out_slice(ry, i),
                dst_ref=out_slice(ry, i),
                send_sem=z2_send_sems.at[i],
                recv_sem=z2_recv_sems.at[i],
                device_id=z_partner,
                device_id_type=pl.DeviceIdType.MESH,
            )

        def z2_wait(i):
            return pltpu.make_async_remote_copy(
                src_ref=out_slice(rd, i),
                dst_ref=out_slice(rd, i),
                send_sem=z2_send_sems.at[i],
                recv_sem=z2_recv_sems.at[i],
                device_id=z_partner,
                device_id_type=pl.DeviceIdType.MESH,
            )

        def y2_rdma(i):
            return pltpu.make_async_remote_copy(
                src_ref=out_slice(rz, i),
                dst_ref=out_slice(rz, i),
                send_sem=y2_send_sems.at[i],
                recv_sem=y2_recv_sems.at[i],
                device_id=y_partner,
                device_id_type=pl.DeviceIdType.MESH,
            )

        def y2_wait(i):
            return pltpu.make_async_remote_copy(
                src_ref=out_slice(rd, i),
                dst_ref=out_slice(rd, i),
                send_sem=y2_send_sems.at[i],
                recv_sem=y2_recv_sems.at[i],
                device_id=y_partner,
                device_id_type=pl.DeviceIdType.MESH,
            )

        def half_partial(row_start):
            acc = None
            for h in range(BH):
                a = o_ref[0, pl.ds(row_start, S_HALF), h, :]
                w = w_ref[pl.ds(h * D, D), :]
                d = jnp.dot(a, w, preferred_element_type=jnp.float32)
                acc = d if acc is None else acc + d
            return acc

        @pl.when(b < NJ)
        def _():
            prod_send = half_partial(send_start)
            prod_keep = half_partial(keep_start)

            @pl.when(k == 0)
            def _():
                send_x[b] = prod_send
                keep[b] = prod_keep

            @pl.when(k != 0)
            def _():
                send_x[b] += prod_send
                keep[b] += prod_keep

            @pl.when(k == NK - 1)
            def _():
                x_rdma(b).start()

        @pl.when(k == NK - 1)
        def _():
            @pl.when((b >= 1) & (b <= NJ))
            def _():
                c = b - 1
                x_rdma(c).wait_recv()
                keep[c] += recv_x[c]
                y_rdma(c).start()
                z1_rdma(c).start()
                pltpu.make_async_copy(
                    keep.at[c], out_slice(r, c), outred_sems.at[c]
                ).start()

            @pl.when((b >= 2) & (b <= NJ + 1))
            def _():
                c2 = b - 2
                y_wait(c2).wait_recv()
                z1_wait(c2).wait_recv()

                @pl.when(c2 % 2 == 0)
                def _():
                    z2_rdma(c2).start()

                @pl.when(c2 % 2 == 1)
                def _():
                    y2_rdma(c2).start()

            @pl.when((b >= 3) & (b <= NJ + 2))
            def _():
                c3 = b - 3

                @pl.when(c3 % 2 == 0)
                def _():
                    z2_wait(c3).wait_recv()

                @pl.when(c3 % 2 == 1)
                def _():
                    y2_wait(c3).wait_recv()

            @pl.when(b == NJ + 2)
            def _():
                for i in range(NJ):
                    x_rdma(i).wait_send()
                    y_rdma(i).wait_send()
                    z1_rdma(i).wait_send()
                    if i % 2 == 0:
                        z2_rdma(i).wait_send()
                    else:
                        y2_rdma(i).wait_send()
                    pltpu.make_async_copy(
                        keep.at[i], out_slice(r, i), outred_sems.at[i]
                    ).wait()

    grid_spec = pltpu.PrefetchScalarGridSpec(
        num_scalar_prefetch=1,
        grid=(NJ + 3, NK),
        in_specs=[
            pl.BlockSpec(
                (1, S, BH, D),
                lambda b, k, r_ref: (0, 0, k, 0),
            ),
            pl.BlockSpec(
                (BK, BN),
                lambda b, k, r_ref: (
                    k,
                    r_ref[0] * NJ + jnp.minimum(b, NJ - 1),
                ),
            ),
        ],
        out_specs=pl.BlockSpec(memory_space=pl.ANY),
        scratch_shapes=[
            pltpu.VMEM((NJ, S_HALF, BN), jnp.float32),
            pltpu.VMEM((NJ, S_HALF, BN), jnp.float32),
            pltpu.VMEM((NJ, S_HALF, BN), jnp.float32),
            pltpu.SemaphoreType.DMA((NJ,)),
            pltpu.SemaphoreType.DMA((NJ,)),
            pltpu.SemaphoreType.DMA((NJ,)),
            pltpu.SemaphoreType.DMA((NJ,)),
            pltpu.SemaphoreType.DMA((NJ,)),
            pltpu.SemaphoreType.DMA((NJ,)),
            pltpu.SemaphoreType.DMA((NJ,)),
            pltpu.SemaphoreType.DMA((NJ,)),
            pltpu.SemaphoreType.DMA((NJ,)),
            pltpu.SemaphoreType.DMA((NJ,)),
            pltpu.SemaphoreType.DMA((NJ,)),
        ],
    )

    out = pl.pallas_call(
        body,
        grid_spec=grid_spec,
        out_shape=jax.ShapeDtypeStruct((1, S_HALF, N), jnp.float32),
        compiler_params=pltpu.CompilerParams(
            vmem_limit_bytes=60 * 1024 * 1024,
        ),
    )(r0, O, Wo)
    return out
